# baseline (speedup 1.0000x reference)
import functools

import jax
import jax.numpy as jnp
from jax import lax
from jax.experimental import pallas as pl
from jax.experimental.pallas import tpu as pltpu
from jax.experimental.pallas import tpu_sc as plsc

_N = 50000
_D = 256
_H = 256
_B = 64
_MAXC = 100
_T = 2 * _B
_BLK = 5000
_NB = _N // _BLK
_NSTREAM = 2
_G = _NB // _NSTREAM

_PREC = jax.lax.Precision.HIGHEST
_PREC_STREAM = jax.lax.Precision.DEFAULT


def _sc_prep_body(feat_hbm, idx_hbm, ta_out, idx_v, rows_v, sem):
    wid = lax.axis_index("s") * 2 + lax.axis_index("c")

    @pl.when(wid < 8)
    def _gather_anchor_rows():
        base = wid * 8
        pltpu.sync_copy(idx_hbm.at[pl.ds(base, 8)], idx_v)
        pltpu.async_copy(feat_hbm.at[idx_v], rows_v, sem).wait()
        pltpu.sync_copy(rows_v, ta_out.at[pl.ds(base, 8)])


def _sc_prep(features, indexes):
    mesh = plsc.VectorSubcoreMesh(core_axis_name="c", subcore_axis_name="s")
    fn = functools.partial(
        pl.kernel, mesh=mesh,
        out_type=jax.ShapeDtypeStruct((_B, _D), jnp.float32),
        scratch_types=[
            pltpu.VMEM((8,), jnp.int32),
            pltpu.VMEM((8, _D), jnp.float32),
            pltpu.SemaphoreType.DMA,
        ],
    )(_sc_prep_body)
    return fn(features, indexes)


def _process_block(f, lbl, blk_idx, ta64, tl, tanch, counts_s, agg_s):
    u64 = jax.lax.dot_general(ta64, f, (((1,), (1,)), ((), ())),
                              preferred_element_type=jnp.float32,
                              precision=_PREC_STREAM)

    iota_row = jax.lax.broadcasted_iota(jnp.int32, (1, _BLK), 1)
    match = (tl == lbl) & ((tanch - blk_idx * _BLK) != iota_row)
    m_f = match.astype(jnp.float32)
    total = jnp.sum(m_f, axis=1, keepdims=True)
    cprev = counts_s[...]

    simple_ok = (cprev + total) <= float(_MAXC)
    u = jnp.concatenate([u64, u64], axis=0)
    w = jnp.where(match & simple_ok, u, 0.0)
    agg_s[...] += jax.lax.dot_general(w, f, (((1,), (0,)), ((), ())),
                                      preferred_element_type=jnp.float32,
                                      precision=_PREC_STREAM)

    crossing = (cprev < float(_MAXC)) & (cprev + total > float(_MAXC))

    @pl.when(jnp.any(crossing))
    def _exact():
        chunk = 1000
        r = jax.lax.broadcasted_iota(jnp.int32, (chunk, chunk), 0)
        c = jax.lax.broadcasted_iota(jnp.int32, (chunk, chunk), 1)
        triu = (r < c).astype(jnp.float32)
        run = cprev
        for s in range(_BLK // chunk):
            sl = slice(s * chunk, (s + 1) * chunk)
            m_c = m_f[:, sl]
            rank = run + jax.lax.dot_general(
                m_c, triu, (((1,), (0,)), ((), ())),
                preferred_element_type=jnp.float32, precision=_PREC)
            take_cx = match[:, sl] & crossing & (rank < float(_MAXC))
            u_c = jnp.concatenate([u64[:, sl], u64[:, sl]], axis=0)
            w_cx = jnp.where(take_cx, u_c, 0.0)
            agg_s[...] += jax.lax.dot_general(
                w_cx, f[sl, :], (((1,), (0,)), ((), ())),
                preferred_element_type=jnp.float32, precision=_PREC)
            run = run + jnp.sum(m_c, axis=1, keepdims=True)

    counts_s[...] = cprev + total


def _main_body(fa_ref, fb_ref, la_ref, lb_ref, ta_ref, tl_ref, tanch_ref,
               tact_ref, wc_ref, bc_ref, w1_ref, b1_ref, pw_ref, w2_ref,
               b2_ref, out_ref, counts_s, agg_s):
    g = pl.program_id(0)

    ta64 = ta_ref[...]

    @pl.when(g == 0)
    def _init():
        n2 = jnp.sum(ta64 * ta64, axis=1, keepdims=True)
        self_term = n2 * ta64
        agg_s[...] = jnp.concatenate([self_term, self_term], axis=0)
        counts_s[...] = jnp.zeros_like(counts_s)

    tl = tl_ref[...]
    tanch = tanch_ref[...]

    _process_block(fa_ref[...], la_ref[0], _NSTREAM * g, ta64, tl, tanch,
                   counts_s, agg_s)
    _process_block(fb_ref[...], lb_ref[0], _NSTREAM * g + 1, ta64, tl, tanch,
                   counts_s, agg_s)

    @pl.when(g == _G - 1)
    def _epilogue():
        ta128 = jnp.concatenate([ta64, ta64], axis=0)
        xin = jnp.concatenate([ta128, agg_s[...]], axis=1)
        h = jnp.maximum(
            jax.lax.dot_general(xin, wc_ref[...], (((1,), (0,)), ((), ())),
                                preferred_element_type=jnp.float32,
                                precision=_PREC) + bc_ref[0, :][None, :], 0.0)
        z = jax.lax.dot_general(h, w1_ref[...], (((1,), (0,)), ((), ())),
                                preferred_element_type=jnp.float32,
                                precision=_PREC) + b1_ref[0, :][None, :]
        pw = pw_ref[0, :][None, :]
        z = jnp.maximum(z, 0.0) + pw * jnp.minimum(z, 0.0)
        lg = jax.lax.dot_general(z, w2_ref[...], (((1,), (0,)), ((), ())),
                                 preferred_element_type=jnp.float32,
                                 precision=_PREC) + b2_ref[0, :][None, :]
        l0 = lg[:, 0:1]
        l1 = lg[:, 1:2]
        m = jnp.maximum(l0, l1)
        term = m + jnp.log(jnp.exp(l0 - m) + jnp.exp(l1 - m)) - l0
        cnt_ok = (counts_s[...] > 0.0).astype(jnp.float32)
        valid = tact_ref[...] * cnt_ok
        loss = jnp.sum(term * valid) * (10.0 / float(_B))
        out_ref[...] = jnp.reshape(loss, (1, 1))


@functools.partial(jax.jit, static_argnames=("interpret",))
def _run(indexes, features, labels, domain, neighbors, all_pred,
         W_conv, b_conv, W1, b1, prelu_w, W2, b2, interpret=False):
    indexes = indexes.astype(jnp.int32)
    labels = labels.astype(jnp.int32)
    neighbors = neighbors.astype(jnp.int32)

    ta64 = _sc_prep(features, indexes)

    lab_nb = labels[neighbors]
    lab_a = lab_nb[:, 0]
    lab_b = lab_nb[:, 1]
    has_b = all_pred[:, 1] >= all_pred[:, 0]
    distinct = has_b & (lab_b != lab_a)
    first = jnp.where(distinct, jnp.minimum(lab_a, lab_b), lab_a)
    second = jnp.maximum(lab_a, lab_b)
    task_labels = jnp.concatenate([first, second], axis=0)
    task_anchor = jnp.concatenate([indexes, indexes], axis=0)
    task_active = jnp.concatenate(
        [jnp.ones((_B,), jnp.float32), distinct.astype(jnp.float32)], axis=0)

    w2p = jnp.zeros((_H, 128), jnp.float32).at[:, :2].set(W2)
    b2p = jnp.zeros((128,), jnp.float32).at[:2].set(b2)

    lbl3 = labels.reshape(_NB, 1, _BLK)

    full = lambda shape: pl.BlockSpec(shape, lambda g: tuple(0 for _ in shape))
    out = pl.pallas_call(
        _main_body,
        grid=(_G,),
        in_specs=[
            pl.BlockSpec((_BLK, _D), lambda g: (_NSTREAM * g, 0)),
            pl.BlockSpec((_BLK, _D), lambda g: (_NSTREAM * g + 1, 0)),
            pl.BlockSpec((1, 1, _BLK), lambda g: (_NSTREAM * g, 0, 0)),
            pl.BlockSpec((1, 1, _BLK), lambda g: (_NSTREAM * g + 1, 0, 0)),
            full((_B, _D)),
            full((_T, 1)),
            full((_T, 1)),
            full((_T, 1)),
            full((2 * _D, _H)),
            full((1, _H)),
            full((_H, _H)),
            full((1, _H)),
            full((1, _H)),
            full((_H, 128)),
            full((1, 128)),
        ],
        out_specs=pl.BlockSpec((1, 1), lambda g: (0, 0)),
        out_shape=jax.ShapeDtypeStruct((1, 1), jnp.float32),
        scratch_shapes=[
            pltpu.VMEM((_T, 1), jnp.float32),
            pltpu.VMEM((_T, _D), jnp.float32),
        ],
        compiler_params=pltpu.CompilerParams(
            dimension_semantics=("arbitrary",)),
        interpret=interpret,
    )(features, features, lbl3, lbl3, ta64,
      task_labels.reshape(_T, 1), task_anchor.reshape(_T, 1),
      task_active.reshape(_T, 1),
      W_conv, b_conv.reshape(1, _H), W1, b1.reshape(1, _H),
      prelu_w.reshape(1, _H), w2p, b2p.reshape(1, 128))

    total = out[0, 0]
    return jnp.where(domain != 0, total, jnp.array(0.0, jnp.float32))


def kernel(indexes, features, labels, domain, neighbors, all_pred,
           W_conv, b_conv, W1, b1, prelu_w, W2, b2):
    return _run(indexes, features, labels, domain, neighbors, all_pred,
                W_conv, b_conv, W1, b1, prelu_w, W2, b2)


# speedup vs baseline: 1.0541x; 1.0541x over previous
import functools

import jax
import jax.numpy as jnp
from jax import lax
from jax.experimental import pallas as pl
from jax.experimental.pallas import tpu as pltpu
from jax.experimental.pallas import tpu_sc as plsc

_N = 50000
_D = 256
_H = 256
_B = 64
_MAXC = 100
_T = 2 * _B
_BLK = 10000
_NB = _N // _BLK
_NSTREAM = 1
_G = _NB // _NSTREAM

_PREC = jax.lax.Precision.HIGHEST
_PREC_STREAM = jax.lax.Precision.DEFAULT


def _sc_prep_body(feat_hbm, idx_hbm, ta_out, idx_v, rows_v, sem):
    wid = lax.axis_index("s") * 2 + lax.axis_index("c")

    @pl.when(wid < 8)
    def _gather_anchor_rows():
        base = wid * 8
        pltpu.sync_copy(idx_hbm.at[pl.ds(base, 8)], idx_v)
        pltpu.async_copy(feat_hbm.at[idx_v], rows_v, sem).wait()
        pltpu.sync_copy(rows_v, ta_out.at[pl.ds(base, 8)])


def _sc_prep(features, indexes):
    mesh = plsc.VectorSubcoreMesh(core_axis_name="c", subcore_axis_name="s")
    fn = functools.partial(
        pl.kernel, mesh=mesh,
        out_type=jax.ShapeDtypeStruct((_B, _D), jnp.float32),
        scratch_types=[
            pltpu.VMEM((8,), jnp.int32),
            pltpu.VMEM((8, _D), jnp.float32),
            pltpu.SemaphoreType.DMA,
        ],
    )(_sc_prep_body)
    return fn(features, indexes)


def _process_block(f, lbl, blk_idx, ta64, tl, tanch, counts_s, agg_s):
    u64 = jax.lax.dot_general(ta64, f, (((1,), (1,)), ((), ())),
                              preferred_element_type=jnp.float32,
                              precision=_PREC_STREAM)

    iota_row = jax.lax.broadcasted_iota(jnp.int32, (1, _BLK), 1)
    match = (tl == lbl) & ((tanch - blk_idx * _BLK) != iota_row)
    m_f = match.astype(jnp.float32)
    total = jnp.sum(m_f, axis=1, keepdims=True)
    cprev = counts_s[...]

    simple_ok = (cprev + total) <= float(_MAXC)
    u = jnp.concatenate([u64, u64], axis=0)
    w = jnp.where(match & simple_ok, u, 0.0)
    agg_s[...] += jax.lax.dot_general(w, f, (((1,), (0,)), ((), ())),
                                      preferred_element_type=jnp.float32,
                                      precision=_PREC_STREAM)

    crossing = (cprev < float(_MAXC)) & (cprev + total > float(_MAXC))

    @pl.when(jnp.any(crossing))
    def _exact():
        chunk = 1000
        r = jax.lax.broadcasted_iota(jnp.int32, (chunk, chunk), 0)
        c = jax.lax.broadcasted_iota(jnp.int32, (chunk, chunk), 1)
        triu = (r < c).astype(jnp.float32)
        run = cprev
        for s in range(_BLK // chunk):
            sl = slice(s * chunk, (s + 1) * chunk)
            m_c = m_f[:, sl]
            rank = run + jax.lax.dot_general(
                m_c, triu, (((1,), (0,)), ((), ())),
                preferred_element_type=jnp.float32, precision=_PREC)
            take_cx = match[:, sl] & crossing & (rank < float(_MAXC))
            u_c = jnp.concatenate([u64[:, sl], u64[:, sl]], axis=0)
            w_cx = jnp.where(take_cx, u_c, 0.0)
            agg_s[...] += jax.lax.dot_general(
                w_cx, f[sl, :], (((1,), (0,)), ((), ())),
                preferred_element_type=jnp.float32, precision=_PREC)
            run = run + jnp.sum(m_c, axis=1, keepdims=True)

    counts_s[...] = cprev + total


def _main_body(fa_ref, la_ref, ta_ref, tl_ref, tanch_ref,
               tact_ref, wc_ref, bc_ref, w1_ref, b1_ref, pw_ref, w2_ref,
               b2_ref, out_ref, counts_s, agg_s):
    g = pl.program_id(0)

    ta64 = ta_ref[...]

    @pl.when(g == 0)
    def _init():
        n2 = jnp.sum(ta64 * ta64, axis=1, keepdims=True)
        self_term = n2 * ta64
        agg_s[...] = jnp.concatenate([self_term, self_term], axis=0)
        counts_s[...] = jnp.zeros_like(counts_s)

    tl = tl_ref[...]
    tanch = tanch_ref[...]

    _process_block(fa_ref[...], la_ref[0], g, ta64, tl, tanch,
                   counts_s, agg_s)

    @pl.when(g == _G - 1)
    def _epilogue():
        ta128 = jnp.concatenate([ta64, ta64], axis=0)
        xin = jnp.concatenate([ta128, agg_s[...]], axis=1)
        h = jnp.maximum(
            jax.lax.dot_general(xin, wc_ref[...], (((1,), (0,)), ((), ())),
                                preferred_element_type=jnp.float32,
                                precision=_PREC) + bc_ref[0, :][None, :], 0.0)
        z = jax.lax.dot_general(h, w1_ref[...], (((1,), (0,)), ((), ())),
                                preferred_element_type=jnp.float32,
                                precision=_PREC) + b1_ref[0, :][None, :]
        pw = pw_ref[0, :][None, :]
        z = jnp.maximum(z, 0.0) + pw * jnp.minimum(z, 0.0)
        lg = jax.lax.dot_general(z, w2_ref[...], (((1,), (0,)), ((), ())),
                                 preferred_element_type=jnp.float32,
                                 precision=_PREC) + b2_ref[0, :][None, :]
        l0 = lg[:, 0:1]
        l1 = lg[:, 1:2]
        m = jnp.maximum(l0, l1)
        term = m + jnp.log(jnp.exp(l0 - m) + jnp.exp(l1 - m)) - l0
        cnt_ok = (counts_s[...] > 0.0).astype(jnp.float32)
        valid = tact_ref[...] * cnt_ok
        loss = jnp.sum(term * valid) * (10.0 / float(_B))
        out_ref[...] = jnp.reshape(loss, (1, 1))


@functools.partial(jax.jit, static_argnames=("interpret",))
def _run(indexes, features, labels, domain, neighbors, all_pred,
         W_conv, b_conv, W1, b1, prelu_w, W2, b2, interpret=False):
    indexes = indexes.astype(jnp.int32)
    labels = labels.astype(jnp.int32)
    neighbors = neighbors.astype(jnp.int32)

    ta64 = _sc_prep(features, indexes)

    lab_nb = labels[neighbors]
    lab_a = lab_nb[:, 0]
    lab_b = lab_nb[:, 1]
    has_b = all_pred[:, 1] >= all_pred[:, 0]
    distinct = has_b & (lab_b != lab_a)
    first = jnp.where(distinct, jnp.minimum(lab_a, lab_b), lab_a)
    second = jnp.maximum(lab_a, lab_b)
    task_labels = jnp.concatenate([first, second], axis=0)
    task_anchor = jnp.concatenate([indexes, indexes], axis=0)
    task_active = jnp.concatenate(
        [jnp.ones((_B,), jnp.float32), distinct.astype(jnp.float32)], axis=0)

    w2p = jnp.zeros((_H, 128), jnp.float32).at[:, :2].set(W2)
    b2p = jnp.zeros((128,), jnp.float32).at[:2].set(b2)

    lbl3 = labels.reshape(_NB, 1, _BLK)

    full = lambda shape: pl.BlockSpec(shape, lambda g: tuple(0 for _ in shape))
    out = pl.pallas_call(
        _main_body,
        grid=(_G,),
        in_specs=[
            pl.BlockSpec((_BLK, _D), lambda g: (g, 0)),
            pl.BlockSpec((1, 1, _BLK), lambda g: (g, 0, 0)),
            full((_B, _D)),
            full((_T, 1)),
            full((_T, 1)),
            full((_T, 1)),
            full((2 * _D, _H)),
            full((1, _H)),
            full((_H, _H)),
            full((1, _H)),
            full((1, _H)),
            full((_H, 128)),
            full((1, 128)),
        ],
        out_specs=pl.BlockSpec((1, 1), lambda g: (0, 0)),
        out_shape=jax.ShapeDtypeStruct((1, 1), jnp.float32),
        scratch_shapes=[
            pltpu.VMEM((_T, 1), jnp.float32),
            pltpu.VMEM((_T, _D), jnp.float32),
        ],
        compiler_params=pltpu.CompilerParams(
            dimension_semantics=("arbitrary",)),
        interpret=interpret,
    )(features, lbl3, ta64,
      task_labels.reshape(_T, 1), task_anchor.reshape(_T, 1),
      task_active.reshape(_T, 1),
      W_conv, b_conv.reshape(1, _H), W1, b1.reshape(1, _H),
      prelu_w.reshape(1, _H), w2p, b2p.reshape(1, 128))

    total = out[0, 0]
    return jnp.where(domain != 0, total, jnp.array(0.0, jnp.float32))


def kernel(indexes, features, labels, domain, neighbors, all_pred,
           W_conv, b_conv, W1, b1, prelu_w, W2, b2):
    return _run(indexes, features, labels, domain, neighbors, all_pred,
                W_conv, b_conv, W1, b1, prelu_w, W2, b2)
